# baseline (speedup 1.0000x reference)
import functools

import jax
import jax.numpy as jnp
from jax import lax
from jax.experimental import pallas as pl
from jax.experimental.pallas import tpu as pltpu
from jax.experimental.pallas import tpu_sc as plsc

N = 10000
D = 128
B = 64
E = 320000

NC = 2
NS = 16
CHUNK = 128
NCHUNKS = 2560
EPAD = NCHUNKS * CHUNK
CPT0 = 160
CPT1 = 0
HCH = 32
NPH0 = CPT0 // HCH
NPH1 = CPT1 // HCH
NROW = 10240
RPT = NROW // NS
DUMMY = N

ROWS_BLK = 1000
NBLK = N // ROWS_BLK


def _sc_segsum_body(table, srcs, dsts, out, acc, src_v, dst_v, rows,
                    sem0, sem1):
    cid = lax.axis_index("c")
    sid = lax.axis_index("s")
    base = jnp.where(cid == 0, sid * CPT0, NS * CPT0 + sid * CPT1)
    nphase = jnp.where(cid == 0, NPH0, NPH1)
    r0 = sid * RPT

    zeros16 = jnp.zeros((16,), jnp.float32)

    @pl.when(cid == 0)
    def _zero():
        with jax.named_scope("zfill"):
            @pl.loop(0, CHUNK)
            def _zrow(i):
                @pl.loop(0, D // 16)
                def _zcol(k):
                    rows[0, i, pl.ds(k * 16, 16)] = zeros16

        with jax.named_scope("zcopy"):
            @pl.loop(0, RPT // CHUNK)
            def _zacc(t):
                pltpu.sync_copy(rows.at[0],
                                acc.at[pl.ds(r0 + t * CHUNK, CHUNK)])

    plsc.subcore_barrier()

    @jax.named_scope("streams")
    @pl.loop(0, nphase)
    def _phase(p):
        cb = base + p * HCH
        pltpu.sync_copy(srcs.at[pl.ds(cb, HCH)], src_v)
        pltpu.sync_copy(dsts.at[pl.ds(cb, HCH)], dst_v)

        pltpu.async_copy(table.at[src_v.at[0]], rows.at[0], sem0)
        pltpu.async_copy(table.at[src_v.at[1]], rows.at[1], sem1)

        @pl.loop(0, HCH - 2, step=2)
        def _step(j):
            pltpu.make_async_copy(table.at[src_v.at[0]], rows.at[0],
                                  sem0).wait()
            pltpu.sync_copy(rows.at[0], acc.at[dst_v.at[j]], add=True)
            pltpu.async_copy(table.at[src_v.at[j + 2]], rows.at[0], sem0)
            pltpu.make_async_copy(table.at[src_v.at[1]], rows.at[1],
                                  sem1).wait()
            pltpu.sync_copy(rows.at[1], acc.at[dst_v.at[j + 1]], add=True)
            pltpu.async_copy(table.at[src_v.at[j + 3]], rows.at[1], sem1)

        pltpu.make_async_copy(table.at[src_v.at[0]], rows.at[0], sem0).wait()
        pltpu.sync_copy(rows.at[0], acc.at[dst_v.at[HCH - 2]], add=True)
        pltpu.make_async_copy(table.at[src_v.at[1]], rows.at[1], sem1).wait()
        pltpu.sync_copy(rows.at[1], acc.at[dst_v.at[HCH - 1]], add=True)

    plsc.subcore_barrier()

    @pl.when(cid == 0)
    def _copyout():
        with jax.named_scope("copyout"):
            pltpu.sync_copy(acc.at[pl.ds(r0, RPT)], out.at[pl.ds(r0, RPT)])


@functools.cache
def _get_segsum():
  return pl.kernel(
    _sc_segsum_body,
    out_type=jax.ShapeDtypeStruct((NROW, D), jnp.float32),
    mesh=plsc.VectorSubcoreMesh(core_axis_name="c", subcore_axis_name="s",
                                num_cores=NC, num_subcores=NS),
    scratch_types=[
        pltpu.VMEM_SHARED((NROW, D), jnp.float32),
        pltpu.VMEM((HCH, CHUNK), jnp.int32),
        pltpu.VMEM((HCH, CHUNK), jnp.int32),
        pltpu.VMEM((2, CHUNK, D), jnp.float32),
        pltpu.SemaphoreType.DMA,
        pltpu.SemaphoreType.DMA,
    ],
  )


def _mlp1_body(p_ref, x_ref, w1_ref, b1_ref, w2_ref, b2_ref, o_ref):
    agg = p_ref[...] + x_ref[...]
    h1 = jnp.maximum(
        jnp.dot(agg, w1_ref[...].T, preferred_element_type=jnp.float32)
        + b1_ref[...], 0.0)
    o_ref[...] = (
        jnp.dot(h1, w2_ref[...].T, preferred_element_type=jnp.float32)
        + b2_ref[...])


def _mlp2_body(p_ref, h_ref, xb_ref, w1_ref, b1_ref, w2_ref, b2_ref,
               wr1_ref, br1_ref, wr2_ref, br2_ref, o_ref, sums, counts):
    i = pl.program_id(0)
    agg = p_ref[...] + h_ref[...]
    t = jnp.maximum(
        jnp.dot(agg, w1_ref[...].T, preferred_element_type=jnp.float32)
        + b1_ref[...], 0.0)
    hb = (jnp.dot(t, w2_ref[...].T, preferred_element_type=jnp.float32)
          + b2_ref[...])

    seg = xb_ref[0]
    ids = lax.broadcasted_iota(jnp.int32, (B, ROWS_BLK), 0)
    onehot = jnp.where(seg == ids, 1.0, 0.0)

    @pl.when(i == 0)
    def _init():
        sums[...] = jnp.zeros_like(sums)
        counts[...] = jnp.zeros_like(counts)

    sums[...] += jnp.dot(onehot, hb, preferred_element_type=jnp.float32)
    cnt = jnp.sum(onehot, axis=1, keepdims=True)
    counts[...] += jnp.broadcast_to(cnt, (B, 128))

    @pl.when(i == pl.num_programs(0) - 1)
    def _finish():
        mean = sums[...] / jnp.maximum(counts[...], 1.0)
        r = jnp.maximum(
            jnp.dot(mean, wr1_ref[...].T, preferred_element_type=jnp.float32)
            + br1_ref[...], 0.0)
        pred = jnp.dot(r, wr2_ref[...].T,
                       preferred_element_type=jnp.float32)
        o_ref[...] = jnp.broadcast_to(pred, (B, 128)) + br2_ref[...]


_W_SPEC = pl.BlockSpec((D, D), lambda i: (0, 0))
_B_SPEC = pl.BlockSpec((1, D), lambda i: (0, 0))

_mlp1 = pl.pallas_call(
    _mlp1_body,
    grid=(NBLK,),
    in_specs=[
        pl.BlockSpec((ROWS_BLK, D), lambda i: (i, 0)),
        pl.BlockSpec((ROWS_BLK, D), lambda i: (i, 0)),
        _W_SPEC, _B_SPEC, _W_SPEC, _B_SPEC,
    ],
    out_specs=pl.BlockSpec((ROWS_BLK, D), lambda i: (i, 0)),
    out_shape=jax.ShapeDtypeStruct((N, D), jnp.float32),
)

_mlp2 = pl.pallas_call(
    _mlp2_body,
    grid=(NBLK,),
    in_specs=[
        pl.BlockSpec((ROWS_BLK, D), lambda i: (i, 0)),
        pl.BlockSpec((ROWS_BLK, D), lambda i: (i, 0)),
        pl.BlockSpec((1, 1, ROWS_BLK), lambda i: (i, 0, 0)),
        _W_SPEC, _B_SPEC, _W_SPEC, _B_SPEC,
        _W_SPEC, _B_SPEC,
        pl.BlockSpec((1, D), lambda i: (0, 0)),
        pl.BlockSpec((1, D), lambda i: (0, 0)),
    ],
    out_specs=pl.BlockSpec((B, 128), lambda i: (0, 0)),
    out_shape=jax.ShapeDtypeStruct((B, 128), jnp.float32),
    scratch_shapes=[
        pltpu.VMEM((B, 128), jnp.float32),
        pltpu.VMEM((B, 128), jnp.float32),
    ],
)


@jax.jit
def kernel(x, edge_index, pos, x_batch,
           W1a, b1a, W2a, b2a, W1b, b1b, W2b, b2b,
           Wr1, br1, Wr2, br2):
    del pos
    pad = EPAD - E
    srcs = jnp.concatenate(
        [edge_index[0], jnp.zeros((pad,), jnp.int32)]).reshape(NCHUNKS, CHUNK)
    dsts = jnp.concatenate(
        [edge_index[1], jnp.full((pad,), DUMMY, jnp.int32)]).reshape(
            NCHUNKS, CHUNK)

    b1a2 = b1a.reshape(1, D)
    b2a2 = b2a.reshape(1, D)
    b1b2 = b1b.reshape(1, D)
    b2b2 = b2b.reshape(1, D)
    br12 = br1.reshape(1, D)
    br22 = jnp.broadcast_to(br2.reshape(1, 1), (1, D))
    xb = x_batch.reshape(NBLK, 1, ROWS_BLK)

    segsum = _get_segsum()
    p1 = segsum(x, srcs, dsts)
    h = _mlp1(p1, x, W1a, b1a2, W2a, b2a2)
    p2 = segsum(h, srcs, dsts)
    out = _mlp2(p2, h, xb, W1b, b1b2, W2b, b2b2,
                Wr1, br12, Wr2, br22)
    return out[:, :1]


# speedup vs baseline: 1.4908x; 1.4908x over previous
import functools

import jax
import jax.numpy as jnp
from jax import lax
from jax.experimental import pallas as pl
from jax.experimental.pallas import tpu as pltpu
from jax.experimental.pallas import tpu_sc as plsc

N = 10000
D = 128
B = 64
E = 320000

NC = 2
NS = 16
CHUNK = 128
NCHUNKS = 2560
EPAD = NCHUNKS * CHUNK
CPT0 = 144
CPT1 = 16
HCH = 16
NPH0 = CPT0 // HCH
NPH1 = CPT1 // HCH
NROW = 10240
RPT = NROW // NS
DUMMY = N

ROWS_BLK = 1000
NBLK = N // ROWS_BLK


def _sc_segsum_body(table, srcs, dsts, out, acc, src_v, dst_v, rows,
                    sem0, sem1):
    cid = lax.axis_index("c")
    sid = lax.axis_index("s")
    base = jnp.where(cid == 0, sid * CPT0, NS * CPT0 + sid * CPT1)
    nphase = jnp.where(cid == 0, NPH0, NPH1)
    r0 = sid * RPT

    zeros16 = jnp.zeros((16,), jnp.float32)

    with jax.named_scope("zfill"):
        @pl.loop(0, CHUNK)
        def _zrow(i):
            @pl.loop(0, D // 16)
            def _zcol(k):
                rows[0, i, pl.ds(k * 16, 16)] = zeros16

    with jax.named_scope("zcopy"):
        @pl.loop(0, RPT // CHUNK)
        def _zacc(t):
            pltpu.sync_copy(rows.at[0], acc.at[pl.ds(r0 + t * CHUNK, CHUNK)])

    plsc.subcore_barrier()

    @jax.named_scope("streams")
    @pl.loop(0, nphase)
    def _phase(p):
        cb = base + p * HCH
        pltpu.sync_copy(srcs.at[pl.ds(cb, HCH)], src_v)
        pltpu.sync_copy(dsts.at[pl.ds(cb, HCH)], dst_v)

        pltpu.async_copy(table.at[src_v.at[0]], rows.at[0], sem0)
        pltpu.async_copy(table.at[src_v.at[1]], rows.at[1], sem1)

        @pl.loop(0, HCH - 2, step=2)
        def _step(j):
            pltpu.make_async_copy(table.at[src_v.at[0]], rows.at[0],
                                  sem0).wait()
            pltpu.sync_copy(rows.at[0], acc.at[dst_v.at[j]], add=True)
            pltpu.async_copy(table.at[src_v.at[j + 2]], rows.at[0], sem0)
            pltpu.make_async_copy(table.at[src_v.at[1]], rows.at[1],
                                  sem1).wait()
            pltpu.sync_copy(rows.at[1], acc.at[dst_v.at[j + 1]], add=True)
            pltpu.async_copy(table.at[src_v.at[j + 3]], rows.at[1], sem1)

        pltpu.make_async_copy(table.at[src_v.at[0]], rows.at[0], sem0).wait()
        pltpu.sync_copy(rows.at[0], acc.at[dst_v.at[HCH - 2]], add=True)
        pltpu.make_async_copy(table.at[src_v.at[1]], rows.at[1], sem1).wait()
        pltpu.sync_copy(rows.at[1], acc.at[dst_v.at[HCH - 1]], add=True)

    plsc.subcore_barrier()
    with jax.named_scope("copyout"):
        pltpu.sync_copy(acc.at[pl.ds(r0, RPT)], out.at[cid, pl.ds(r0, RPT)])


@functools.cache
def _get_segsum():
  return pl.kernel(
    _sc_segsum_body,
    out_type=jax.ShapeDtypeStruct((NC, NROW, D), jnp.float32),
    mesh=plsc.VectorSubcoreMesh(core_axis_name="c", subcore_axis_name="s",
                                num_cores=NC, num_subcores=NS),
    scratch_types=[
        pltpu.VMEM_SHARED((NROW, D), jnp.float32),
        pltpu.VMEM((HCH, CHUNK), jnp.int32),
        pltpu.VMEM((HCH, CHUNK), jnp.int32),
        pltpu.VMEM((2, CHUNK, D), jnp.float32),
        pltpu.SemaphoreType.DMA,
        pltpu.SemaphoreType.DMA,
    ],
  )


def _mlp1_body(p_ref, x_ref, w1_ref, b1_ref, w2_ref, b2_ref, o_ref):
    agg = p_ref[0] + p_ref[1] + x_ref[...]
    h1 = jnp.maximum(
        jnp.dot(agg, w1_ref[...].T, preferred_element_type=jnp.float32)
        + b1_ref[...], 0.0)
    o_ref[...] = (
        jnp.dot(h1, w2_ref[...].T, preferred_element_type=jnp.float32)
        + b2_ref[...])


def _mlp2_body(p_ref, h_ref, xb_ref, w1_ref, b1_ref, w2_ref, b2_ref,
               wr1_ref, br1_ref, wr2_ref, br2_ref, o_ref, sums, counts):
    i = pl.program_id(0)
    agg = p_ref[0] + p_ref[1] + h_ref[...]
    t = jnp.maximum(
        jnp.dot(agg, w1_ref[...].T, preferred_element_type=jnp.float32)
        + b1_ref[...], 0.0)
    hb = (jnp.dot(t, w2_ref[...].T, preferred_element_type=jnp.float32)
          + b2_ref[...])

    seg = xb_ref[0]
    ids = lax.broadcasted_iota(jnp.int32, (B, ROWS_BLK), 0)
    onehot = jnp.where(seg == ids, 1.0, 0.0)

    @pl.when(i == 0)
    def _init():
        sums[...] = jnp.zeros_like(sums)
        counts[...] = jnp.zeros_like(counts)

    sums[...] += jnp.dot(onehot, hb, preferred_element_type=jnp.float32)
    cnt = jnp.sum(onehot, axis=1, keepdims=True)
    counts[...] += jnp.broadcast_to(cnt, (B, 128))

    @pl.when(i == pl.num_programs(0) - 1)
    def _finish():
        mean = sums[...] / jnp.maximum(counts[...], 1.0)
        r = jnp.maximum(
            jnp.dot(mean, wr1_ref[...].T, preferred_element_type=jnp.float32)
            + br1_ref[...], 0.0)
        pred = jnp.dot(r, wr2_ref[...].T,
                       preferred_element_type=jnp.float32)
        o_ref[...] = jnp.broadcast_to(pred, (B, 128)) + br2_ref[...]


_W_SPEC = pl.BlockSpec((D, D), lambda i: (0, 0))
_B_SPEC = pl.BlockSpec((1, D), lambda i: (0, 0))

_mlp1 = pl.pallas_call(
    _mlp1_body,
    grid=(NBLK,),
    in_specs=[
        pl.BlockSpec((NC, ROWS_BLK, D), lambda i: (0, i, 0)),
        pl.BlockSpec((ROWS_BLK, D), lambda i: (i, 0)),
        _W_SPEC, _B_SPEC, _W_SPEC, _B_SPEC,
    ],
    out_specs=pl.BlockSpec((ROWS_BLK, D), lambda i: (i, 0)),
    out_shape=jax.ShapeDtypeStruct((N, D), jnp.float32),
)

_mlp2 = pl.pallas_call(
    _mlp2_body,
    grid=(NBLK,),
    in_specs=[
        pl.BlockSpec((NC, ROWS_BLK, D), lambda i: (0, i, 0)),
        pl.BlockSpec((ROWS_BLK, D), lambda i: (i, 0)),
        pl.BlockSpec((1, 1, ROWS_BLK), lambda i: (i, 0, 0)),
        _W_SPEC, _B_SPEC, _W_SPEC, _B_SPEC,
        _W_SPEC, _B_SPEC,
        pl.BlockSpec((1, D), lambda i: (0, 0)),
        pl.BlockSpec((1, D), lambda i: (0, 0)),
    ],
    out_specs=pl.BlockSpec((B, 128), lambda i: (0, 0)),
    out_shape=jax.ShapeDtypeStruct((B, 128), jnp.float32),
    scratch_shapes=[
        pltpu.VMEM((B, 128), jnp.float32),
        pltpu.VMEM((B, 128), jnp.float32),
    ],
)


@jax.jit
def kernel(x, edge_index, pos, x_batch,
           W1a, b1a, W2a, b2a, W1b, b1b, W2b, b2b,
           Wr1, br1, Wr2, br2):
    del pos
    pad = EPAD - E
    srcs = jnp.concatenate(
        [edge_index[0], jnp.zeros((pad,), jnp.int32)]).reshape(NCHUNKS, CHUNK)
    dsts = jnp.concatenate(
        [edge_index[1], jnp.full((pad,), DUMMY, jnp.int32)]).reshape(
            NCHUNKS, CHUNK)

    b1a2 = b1a.reshape(1, D)
    b2a2 = b2a.reshape(1, D)
    b1b2 = b1b.reshape(1, D)
    b2b2 = b2b.reshape(1, D)
    br12 = br1.reshape(1, D)
    br22 = jnp.broadcast_to(br2.reshape(1, 1), (1, D))
    xb = x_batch.reshape(NBLK, 1, ROWS_BLK)

    segsum = _get_segsum()
    p1 = segsum(x, srcs, dsts)
    h = _mlp1(p1, x, W1a, b1a2, W2a, b2a2)
    p2 = segsum(h, srcs, dsts)
    out = _mlp2(p2, h, xb, W1b, b1b2, W2b, b2b2,
                Wr1, br12, Wr2, br22)
    return out[:, :1]
